# baseline (speedup 1.0000x reference)
import functools

import jax
import jax.numpy as jnp
from jax import lax
from jax.experimental import pallas as pl
from jax.experimental.pallas import tpu as pltpu
from jax.experimental.pallas import tpu_sc as plsc

_NC = 2
_NS = 16
_NW = _NC * _NS
_CH = 128
_NP = 10240
_BM = 2000
_U = 10
_STEPS = 80
_EPAD = _NW * _STEPS * _CH


@functools.lru_cache(maxsize=None)
def _make_sc_segsum(D):
  rpt = _NP // _NS
  mesh = plsc.VectorSubcoreMesh(core_axis_name="c", subcore_axis_name="s")
  out_type = jax.ShapeDtypeStruct((_NC, _NP, D), jnp.float32)
  scratch = (pltpu.VMEM_SHARED((_NP, D), jnp.float32),
             pltpu.VMEM((_U, _CH), jnp.int32),
             pltpu.VMEM((_U, _CH), jnp.int32),
             pltpu.VMEM((_CH, D), jnp.float32),
             pltpu.VMEM((_CH, D), jnp.float32),
             pltpu.SemaphoreType.DMA,
             pltpu.SemaphoreType.DMA,
             pltpu.SemaphoreType.DMA,
             pltpu.SemaphoreType.DMA,
             pltpu.SemaphoreType.DMA)

  def body(y_h, src_h, dst_h, zD_h, agg_h,
           acc, si, di, rows0, rows1, semis, semid, semg, sems0, sems1):
    rows = (rows0, rows1)
    sems = (sems0, sems1)
    c = lax.axis_index("c")
    s = lax.axis_index("s")
    wid = c * _NS + s
    r0 = s * rpt
    pltpu.sync_copy(zD_h.at[pl.ds(r0, rpt)], acc.at[pl.ds(r0, rpt)])
    plsc.subcore_barrier()
    base = wid * _STEPS * _CH

    def outer(t, carry):
      bb = pl.multiple_of(base + t * _U * _CH, 8)
      idx_descs = []
      for k in range(_U):
        off = pl.multiple_of(bb + k * _CH, 8)
        idx_descs.append(
            pltpu.async_copy(src_h.at[pl.ds(off, _CH)], si.at[k], semis))
        idx_descs.append(
            pltpu.async_copy(dst_h.at[pl.ds(off, _CH)], di.at[k], semid))
      for d in idx_descs:
        d.wait()
      scat = [None, None]
      for k in range(_U):
        b2 = k % 2
        pltpu.async_copy(y_h.at[si.at[k]], rows[b2], semg).wait()
        if k >= 1:
          scat[1 - b2].wait()
        scat[b2] = pltpu.async_copy(rows[b2], acc.at[di.at[k]],
                                    sems[b2], add=True)
      scat[(_U - 1) % 2].wait()
      return carry

    lax.fori_loop(0, _STEPS // _U, outer, 0)
    plsc.subcore_barrier()
    pltpu.sync_copy(acc.at[pl.ds(r0, rpt)], agg_h.at[c, pl.ds(r0, rpt)])

  return pl.kernel(body, out_type=out_type, mesh=mesh, scratch_types=scratch)


@functools.lru_cache(maxsize=None)
def _make_sc_degree():
  rpt = _NP // _NS
  mesh = plsc.VectorSubcoreMesh(core_axis_name="c", subcore_axis_name="s")
  out_type = jax.ShapeDtypeStruct((_NC, _NP, 128), jnp.float32)
  scratch = (pltpu.VMEM_SHARED((_NP, 128), jnp.float32),
             pltpu.VMEM((_CH, 128), jnp.float32),
             pltpu.VMEM((_U, _CH), jnp.int32),
             pltpu.SemaphoreType.DMA,
             pltpu.SemaphoreType.DMA,
             pltpu.SemaphoreType.DMA)

  def body(dst_h, zD_h, ones_h, cnt_h, acc, ones, di, semid, sems0, sems1):
    sems = (sems0, sems1)
    c = lax.axis_index("c")
    s = lax.axis_index("s")
    wid = c * _NS + s
    r0 = s * rpt
    pltpu.sync_copy(zD_h.at[pl.ds(r0, rpt)], acc.at[pl.ds(r0, rpt)])
    pltpu.sync_copy(ones_h, ones)
    plsc.subcore_barrier()
    base = wid * _STEPS * _CH

    def outer(t, carry):
      bb = pl.multiple_of(base + t * _U * _CH, 8)
      idx_descs = []
      for k in range(_U):
        off = pl.multiple_of(bb + k * _CH, 8)
        idx_descs.append(
            pltpu.async_copy(dst_h.at[pl.ds(off, _CH)], di.at[k], semid))
      for d in idx_descs:
        d.wait()
      scat = [None, None]
      for k in range(_U):
        b2 = k % 2
        if k >= 2:
          scat[b2].wait()
        scat[b2] = pltpu.async_copy(ones, acc.at[di.at[k]],
                                    sems[b2], add=True)
      scat[0].wait()
      scat[1].wait()
      return carry

    lax.fori_loop(0, _STEPS // _U, outer, 0)
    plsc.subcore_barrier()
    pltpu.sync_copy(acc.at[pl.ds(r0, rpt)], cnt_h.at[c, pl.ds(r0, rpt)])

  return pl.kernel(body, out_type=out_type, mesh=mesh, scratch_types=scratch)


def _pad_edges(idx, fill):
  return jnp.concatenate(
      [idx, jnp.full((_EPAD - idx.shape[0],), fill, jnp.int32)])


def _sc_segsum(y, src_p, dst_p):
  _, D = y.shape
  k = _make_sc_segsum(D)
  zD = jnp.zeros((_NP, D), jnp.float32)
  return k(y, src_p, dst_p, zD)


def _sc_degree(dst_p):
  k = _make_sc_degree()
  zD = jnp.zeros((_NP, 128), jnp.float32)
  ones = jnp.ones((_CH, 128), jnp.float32)
  return k(dst_p, zD, ones)


def _tc_proj2(x, Wl, Wr, bl):
  N, Din = x.shape
  Dl, Dr = Wl.shape[1], Wr.shape[1]

  def body(x_ref, wl_ref, wr_ref, b_ref, y_ref, z_ref):
    xb = x_ref[...]
    y_ref[...] = jnp.dot(xb, wl_ref[...], preferred_element_type=jnp.float32)
    z_ref[...] = (jnp.dot(xb, wr_ref[...], preferred_element_type=jnp.float32)
                  + b_ref[...])

  return pl.pallas_call(
      body,
      grid=(N // _BM,),
      in_specs=[pl.BlockSpec((_BM, Din), lambda i: (i, 0)),
                pl.BlockSpec((Din, Dl), lambda i: (0, 0)),
                pl.BlockSpec((Din, Dr), lambda i: (0, 0)),
                pl.BlockSpec((1, Dr), lambda i: (0, 0))],
      out_specs=[pl.BlockSpec((_BM, Dl), lambda i: (i, 0)),
                 pl.BlockSpec((_BM, Dr), lambda i: (i, 0))],
      out_shape=[jax.ShapeDtypeStruct((N, Dl), jnp.float32),
                 jax.ShapeDtypeStruct((N, Dr), jnp.float32)],
  )(x, Wl, Wr, bl.reshape(1, Dr))


def _tc_combine_proj(aggp, cntp, z1, Wl, Wr, b):
  N, H = z1.shape
  Do = Wl.shape[1]

  def body(a_ref, c_ref, z_ref, wl_ref, wr_ref, b_ref, y_ref, z2_ref):
    a = a_ref[0] + a_ref[1]
    cnt = c_ref[0, :, 0:1] + c_ref[1, :, 0:1]
    inv = 1.0 / jnp.maximum(cnt, 1.0)
    h = jnp.maximum(a * inv + z_ref[...], 0.0)
    y_ref[...] = jnp.dot(h, wl_ref[...], preferred_element_type=jnp.float32)
    z2_ref[...] = (jnp.dot(h, wr_ref[...], preferred_element_type=jnp.float32)
                   + b_ref[...])

  return pl.pallas_call(
      body,
      grid=(N // _BM,),
      in_specs=[pl.BlockSpec((2, _BM, H), lambda i: (0, i, 0)),
                pl.BlockSpec((2, _BM, 128), lambda i: (0, i, 0)),
                pl.BlockSpec((_BM, H), lambda i: (i, 0)),
                pl.BlockSpec((H, Do), lambda i: (0, 0)),
                pl.BlockSpec((H, Do), lambda i: (0, 0)),
                pl.BlockSpec((1, Do), lambda i: (0, 0))],
      out_specs=[pl.BlockSpec((_BM, Do), lambda i: (i, 0)),
                 pl.BlockSpec((_BM, Do), lambda i: (i, 0))],
      out_shape=[jax.ShapeDtypeStruct((N, Do), jnp.float32),
                 jax.ShapeDtypeStruct((N, Do), jnp.float32)],
  )(aggp, cntp, z1, Wl, Wr, b.reshape(1, Do))


def _tc_final(aggp, cntp, z2):
  N, Do = z2.shape

  def body(a_ref, c_ref, z_ref, o_ref):
    a = a_ref[0] + a_ref[1]
    cnt = c_ref[0, :, 0:1] + c_ref[1, :, 0:1]
    inv = 1.0 / jnp.maximum(cnt, 1.0)
    o_ref[...] = a * inv + z_ref[...]

  return pl.pallas_call(
      body,
      grid=(N // _BM,),
      in_specs=[pl.BlockSpec((2, _BM, Do), lambda i: (0, i, 0)),
                pl.BlockSpec((2, _BM, 128), lambda i: (0, i, 0)),
                pl.BlockSpec((_BM, Do), lambda i: (i, 0))],
      out_specs=pl.BlockSpec((_BM, Do), lambda i: (i, 0)),
      out_shape=jax.ShapeDtypeStruct((N, Do), jnp.float32),
  )(aggp, cntp, z2)


def kernel(x, edge_index, W1l, b1l, W1r, W2l, b2l, W2r):
  N, _ = x.shape
  C = W2l.shape[1]
  Dp = 128
  src = _pad_edges(edge_index[0].astype(jnp.int32), 0)
  dst = _pad_edges(edge_index[1].astype(jnp.int32), N)

  cntp = _sc_degree(dst)
  y1, z1 = _tc_proj2(x, W1l, W1r, b1l)
  agg1p = _sc_segsum(y1, src, dst)

  W2l_p = jnp.zeros((W2l.shape[0], Dp), jnp.float32).at[:, :C].set(W2l)
  W2r_p = jnp.zeros((W2r.shape[0], Dp), jnp.float32).at[:, :C].set(W2r)
  b2_p = jnp.zeros((Dp,), jnp.float32).at[:C].set(b2l)

  y2, z2 = _tc_combine_proj(agg1p, cntp, z1, W2l_p, W2r_p, b2_p)
  agg2p = _sc_segsum(y2, src, dst)
  out = _tc_final(agg2p, cntp, z2)
  return out[:, :C]


# speedup vs baseline: 2.2776x; 2.2776x over previous
import functools

import jax
import jax.numpy as jnp
from jax import lax
from jax.experimental import pallas as pl
from jax.experimental.pallas import tpu as pltpu
from jax.experimental.pallas import tpu_sc as plsc

_NC = 2
_NS = 16
_NW = _NC * _NS
_CH = 128
_NP = 10240
_BM = 2000
_U = 10
_STEPS = 80
_EPAD = _NW * _STEPS * _CH


@functools.lru_cache(maxsize=None)
def _make_sc_segsum(D):
  rpt = _NP // _NS
  mesh = plsc.VectorSubcoreMesh(core_axis_name="c", subcore_axis_name="s")
  out_type = jax.ShapeDtypeStruct((_NC, _NP, D), jnp.float32)
  scratch = (pltpu.VMEM_SHARED((_NP, D), jnp.float32),
             pltpu.VMEM((_U, _CH), jnp.int32),
             pltpu.VMEM((_U, _CH), jnp.int32),
             pltpu.VMEM((_CH, D), jnp.float32),
             pltpu.VMEM((_CH, D), jnp.float32),
             pltpu.SemaphoreType.DMA,
             pltpu.SemaphoreType.DMA,
             pltpu.SemaphoreType.DMA,
             pltpu.SemaphoreType.DMA,
             pltpu.SemaphoreType.DMA)

  def body(y_h, src_h, dst_h, zD_h, agg_h,
           acc, si, di, rows0, rows1, semis, semid, semg, sems0, sems1):
    rows = (rows0, rows1)
    sems = (sems0, sems1)
    c = lax.axis_index("c")
    s = lax.axis_index("s")
    wid = c * _NS + s
    r0 = s * rpt
    pltpu.sync_copy(zD_h.at[pl.ds(r0, rpt)], acc.at[pl.ds(r0, rpt)])
    plsc.subcore_barrier()
    base = wid * _STEPS * _CH

    def outer(t, carry):
      bb = pl.multiple_of(base + t * _U * _CH, 8)
      idx_descs = []
      for k in range(_U):
        off = pl.multiple_of(bb + k * _CH, 8)
        idx_descs.append(
            pltpu.async_copy(src_h.at[pl.ds(off, _CH)], si.at[k], semis))
        idx_descs.append(
            pltpu.async_copy(dst_h.at[pl.ds(off, _CH)], di.at[k], semid))
      for d in idx_descs:
        d.wait()
      scat = [None, None]
      for k in range(_U):
        b2 = k % 2
        pltpu.async_copy(y_h.at[si.at[k]], rows[b2], semg).wait()
        if k >= 1:
          scat[1 - b2].wait()
        scat[b2] = pltpu.async_copy(rows[b2], acc.at[di.at[k]],
                                    sems[b2], add=True)
      scat[(_U - 1) % 2].wait()
      return carry

    lax.fori_loop(0, _STEPS // _U, outer, 0)
    plsc.subcore_barrier()
    pltpu.sync_copy(acc.at[pl.ds(r0, rpt)], agg_h.at[c, pl.ds(r0, rpt)])

  return pl.kernel(body, out_type=out_type, mesh=mesh, scratch_types=scratch)


@functools.lru_cache(maxsize=None)
def _make_sc_degree():
  rpt = _NP // _NS
  mesh = plsc.VectorSubcoreMesh(core_axis_name="c", subcore_axis_name="s")
  out_type = jax.ShapeDtypeStruct((_NC, _NP, 128), jnp.float32)
  scratch = (pltpu.VMEM_SHARED((_NP, 128), jnp.float32),
             pltpu.VMEM((_CH, 128), jnp.float32),
             pltpu.VMEM((_U, _CH), jnp.int32),
             pltpu.SemaphoreType.DMA,
             pltpu.SemaphoreType.DMA,
             pltpu.SemaphoreType.DMA)

  def body(dst_h, zD_h, ones_h, cnt_h, acc, ones, di, semid, sems0, sems1):
    sems = (sems0, sems1)
    c = lax.axis_index("c")
    s = lax.axis_index("s")
    wid = c * _NS + s
    r0 = s * rpt
    pltpu.sync_copy(zD_h.at[pl.ds(r0, rpt)], acc.at[pl.ds(r0, rpt)])
    pltpu.sync_copy(ones_h, ones)
    plsc.subcore_barrier()
    base = wid * _STEPS * _CH

    def outer(t, carry):
      bb = pl.multiple_of(base + t * _U * _CH, 8)
      idx_descs = []
      for k in range(_U):
        off = pl.multiple_of(bb + k * _CH, 8)
        idx_descs.append(
            pltpu.async_copy(dst_h.at[pl.ds(off, _CH)], di.at[k], semid))
      for d in idx_descs:
        d.wait()
      scat = [None, None]
      for k in range(_U):
        b2 = k % 2
        if k >= 2:
          scat[b2].wait()
        scat[b2] = pltpu.async_copy(ones, acc.at[di.at[k]],
                                    sems[b2], add=True)
      scat[0].wait()
      scat[1].wait()
      return carry

    lax.fori_loop(0, _STEPS // _U, outer, 0)
    plsc.subcore_barrier()
    pltpu.sync_copy(acc.at[pl.ds(r0, rpt)], cnt_h.at[c, pl.ds(r0, rpt)])

  return pl.kernel(body, out_type=out_type, mesh=mesh, scratch_types=scratch)


def _pad_edges(idx, base, mod):
  npad = _EPAD - idx.shape[0]
  fill = base + jnp.arange(npad, dtype=jnp.int32) % mod
  return jnp.concatenate([idx, fill])


def _sc_segsum(y, src_p, dst_p):
  _, D = y.shape
  k = _make_sc_segsum(D)
  zD = jnp.zeros((_NP, D), jnp.float32)
  return k(y, src_p, dst_p, zD)


def _sc_degree(dst_p):
  k = _make_sc_degree()
  zD = jnp.zeros((_NP, 128), jnp.float32)
  ones = jnp.ones((_CH, 128), jnp.float32)
  return k(dst_p, zD, ones)


def _tc_proj2(x, Wl, Wr, bl):
  N, Din = x.shape
  Dl, Dr = Wl.shape[1], Wr.shape[1]

  def body(x_ref, wl_ref, wr_ref, b_ref, y_ref, z_ref):
    xb = x_ref[...]
    y_ref[...] = jnp.dot(xb, wl_ref[...], preferred_element_type=jnp.float32)
    z_ref[...] = (jnp.dot(xb, wr_ref[...], preferred_element_type=jnp.float32)
                  + b_ref[...])

  return pl.pallas_call(
      body,
      grid=(N // _BM,),
      in_specs=[pl.BlockSpec((_BM, Din), lambda i: (i, 0)),
                pl.BlockSpec((Din, Dl), lambda i: (0, 0)),
                pl.BlockSpec((Din, Dr), lambda i: (0, 0)),
                pl.BlockSpec((1, Dr), lambda i: (0, 0))],
      out_specs=[pl.BlockSpec((_BM, Dl), lambda i: (i, 0)),
                 pl.BlockSpec((_BM, Dr), lambda i: (i, 0))],
      out_shape=[jax.ShapeDtypeStruct((N, Dl), jnp.float32),
                 jax.ShapeDtypeStruct((N, Dr), jnp.float32)],
  )(x, Wl, Wr, bl.reshape(1, Dr))


def _tc_combine_proj(aggp, cntp, z1, Wl, Wr, b):
  N, H = z1.shape
  Do = Wl.shape[1]

  def body(a_ref, c_ref, z_ref, wl_ref, wr_ref, b_ref, y_ref, z2_ref):
    a = a_ref[0] + a_ref[1]
    cnt = c_ref[0, :, 0:1] + c_ref[1, :, 0:1]
    inv = 1.0 / jnp.maximum(cnt, 1.0)
    h = jnp.maximum(a * inv + z_ref[...], 0.0)
    y_ref[...] = jnp.dot(h, wl_ref[...], preferred_element_type=jnp.float32)
    z2_ref[...] = (jnp.dot(h, wr_ref[...], preferred_element_type=jnp.float32)
                   + b_ref[...])

  return pl.pallas_call(
      body,
      grid=(N // _BM,),
      in_specs=[pl.BlockSpec((2, _BM, H), lambda i: (0, i, 0)),
                pl.BlockSpec((2, _BM, 128), lambda i: (0, i, 0)),
                pl.BlockSpec((_BM, H), lambda i: (i, 0)),
                pl.BlockSpec((H, Do), lambda i: (0, 0)),
                pl.BlockSpec((H, Do), lambda i: (0, 0)),
                pl.BlockSpec((1, Do), lambda i: (0, 0))],
      out_specs=[pl.BlockSpec((_BM, Do), lambda i: (i, 0)),
                 pl.BlockSpec((_BM, Do), lambda i: (i, 0))],
      out_shape=[jax.ShapeDtypeStruct((N, Do), jnp.float32),
                 jax.ShapeDtypeStruct((N, Do), jnp.float32)],
  )(aggp, cntp, z1, Wl, Wr, b.reshape(1, Do))


def _tc_final(aggp, cntp, z2):
  N, Do = z2.shape

  def body(a_ref, c_ref, z_ref, o_ref):
    a = a_ref[0] + a_ref[1]
    cnt = c_ref[0, :, 0:1] + c_ref[1, :, 0:1]
    inv = 1.0 / jnp.maximum(cnt, 1.0)
    o_ref[...] = a * inv + z_ref[...]

  return pl.pallas_call(
      body,
      grid=(N // _BM,),
      in_specs=[pl.BlockSpec((2, _BM, Do), lambda i: (0, i, 0)),
                pl.BlockSpec((2, _BM, 128), lambda i: (0, i, 0)),
                pl.BlockSpec((_BM, Do), lambda i: (i, 0))],
      out_specs=pl.BlockSpec((_BM, Do), lambda i: (i, 0)),
      out_shape=jax.ShapeDtypeStruct((N, Do), jnp.float32),
  )(aggp, cntp, z2)


def kernel(x, edge_index, W1l, b1l, W1r, W2l, b2l, W2r):
  N, _ = x.shape
  C = W2l.shape[1]
  Dp = 128
  src = _pad_edges(edge_index[0].astype(jnp.int32), 0, _CH)
  dst = _pad_edges(edge_index[1].astype(jnp.int32), N, _NP - N)

  cntp = _sc_degree(dst)
  y1, z1 = _tc_proj2(x, W1l, W1r, b1l)
  agg1p = _sc_segsum(y1, src, dst)

  W2l_p = jnp.zeros((W2l.shape[0], Dp), jnp.float32).at[:, :C].set(W2l)
  W2r_p = jnp.zeros((W2r.shape[0], Dp), jnp.float32).at[:, :C].set(W2r)
  b2_p = jnp.zeros((Dp,), jnp.float32).at[:C].set(b2l)

  y2, z2 = _tc_combine_proj(agg1p, cntp, z1, W2l_p, W2r_p, b2_p)
  agg2p = _sc_segsum(y2, src, dst)
  out = _tc_final(agg2p, cntp, z2)
  return out[:, :C]


# speedup vs baseline: 2.4343x; 1.0688x over previous
import functools

import jax
import jax.numpy as jnp
from jax import lax
from jax.experimental import pallas as pl
from jax.experimental.pallas import tpu as pltpu
from jax.experimental.pallas import tpu_sc as plsc

_NC = 2
_NS = 16
_NW = _NC * _NS
_CH = 128
_NP = 10240
_BM = 2000
_U = 10
_STEPS = 80
_EPAD = _NW * _STEPS * _CH


@functools.lru_cache(maxsize=None)
def _make_sc_segsum(D, tc_tiling=True):
  rpt = _NP // _NS
  mesh = plsc.VectorSubcoreMesh(core_axis_name="c", subcore_axis_name="s")
  out_type = jax.ShapeDtypeStruct((_NC, _NP, D), jnp.float32)
  scratch = (pltpu.VMEM_SHARED((_NP, D), jnp.float32),
             pltpu.VMEM((_U, _CH), jnp.int32),
             pltpu.VMEM((_U, _CH), jnp.int32),
             pltpu.VMEM((_CH, D), jnp.float32),
             pltpu.VMEM((_CH, D), jnp.float32),
             pltpu.SemaphoreType.DMA,
             pltpu.SemaphoreType.DMA,
             pltpu.SemaphoreType.DMA,
             pltpu.SemaphoreType.DMA,
             pltpu.SemaphoreType.DMA)

  def body(y_h, src_h, dst_h, zD_h, agg_h,
           acc, si, di, rows0, rows1, semis, semid, semg, sems0, sems1):
    rows = (rows0, rows1)
    sems = (sems0, sems1)
    c = lax.axis_index("c")
    s = lax.axis_index("s")
    wid = c * _NS + s
    r0 = s * rpt
    pltpu.sync_copy(zD_h.at[pl.ds(r0, rpt)], acc.at[pl.ds(r0, rpt)])
    plsc.subcore_barrier()
    base = wid * _STEPS * _CH

    def outer(t, carry):
      bb = pl.multiple_of(base + t * _U * _CH, 8)
      idx_descs = []
      for k in range(_U):
        off = pl.multiple_of(bb + k * _CH, 8)
        idx_descs.append(
            pltpu.async_copy(src_h.at[pl.ds(off, _CH)], si.at[k], semis))
        idx_descs.append(
            pltpu.async_copy(dst_h.at[pl.ds(off, _CH)], di.at[k], semid))
      for d in idx_descs:
        d.wait()
      scat = [None, None]
      for k in range(_U):
        b2 = k % 2
        pltpu.async_copy(y_h.at[si.at[k]], rows[b2], semg).wait()
        if k >= 1:
          scat[1 - b2].wait()
        scat[b2] = pltpu.async_copy(rows[b2], acc.at[di.at[k]],
                                    sems[b2], add=True)
      scat[(_U - 1) % 2].wait()
      return carry

    lax.fori_loop(0, _STEPS // _U, outer, 0)
    plsc.subcore_barrier()
    pltpu.sync_copy(acc.at[pl.ds(r0, rpt)], agg_h.at[c, pl.ds(r0, rpt)])

  cp = None if tc_tiling else pltpu.CompilerParams(use_tc_tiling_on_sc=False)
  return pl.kernel(body, out_type=out_type, mesh=mesh, scratch_types=scratch,
                   compiler_params=cp)


@functools.lru_cache(maxsize=None)
def _make_sc_degree():
  rpt = _NP // _NS
  mesh = plsc.VectorSubcoreMesh(core_axis_name="c", subcore_axis_name="s")
  out_type = jax.ShapeDtypeStruct((_NC, _NP, 128), jnp.float32)
  scratch = (pltpu.VMEM_SHARED((_NP, 128), jnp.float32),
             pltpu.VMEM((_CH, 128), jnp.float32),
             pltpu.VMEM((_U, _CH), jnp.int32),
             pltpu.SemaphoreType.DMA,
             pltpu.SemaphoreType.DMA,
             pltpu.SemaphoreType.DMA)

  def body(dst_h, zD_h, ones_h, cnt_h, acc, ones, di, semid, sems0, sems1):
    sems = (sems0, sems1)
    c = lax.axis_index("c")
    s = lax.axis_index("s")
    wid = c * _NS + s
    r0 = s * rpt
    pltpu.sync_copy(zD_h.at[pl.ds(r0, rpt)], acc.at[pl.ds(r0, rpt)])
    pltpu.sync_copy(ones_h, ones)
    plsc.subcore_barrier()
    base = wid * _STEPS * _CH

    def outer(t, carry):
      bb = pl.multiple_of(base + t * _U * _CH, 8)
      idx_descs = []
      for k in range(_U):
        off = pl.multiple_of(bb + k * _CH, 8)
        idx_descs.append(
            pltpu.async_copy(dst_h.at[pl.ds(off, _CH)], di.at[k], semid))
      for d in idx_descs:
        d.wait()
      scat = [None, None]
      for k in range(_U):
        b2 = k % 2
        if k >= 2:
          scat[b2].wait()
        scat[b2] = pltpu.async_copy(ones, acc.at[di.at[k]],
                                    sems[b2], add=True)
      scat[0].wait()
      scat[1].wait()
      return carry

    lax.fori_loop(0, _STEPS // _U, outer, 0)
    plsc.subcore_barrier()
    pltpu.sync_copy(acc.at[pl.ds(r0, rpt)], cnt_h.at[c, pl.ds(r0, rpt)])

  return pl.kernel(body, out_type=out_type, mesh=mesh, scratch_types=scratch)


def _pad_edges(idx, base, mod):
  npad = _EPAD - idx.shape[0]
  fill = base + jnp.arange(npad, dtype=jnp.int32) % mod
  return jnp.concatenate([idx, fill])


def _sc_segsum(y, src_p, dst_p):
  _, D = y.shape
  k = _make_sc_segsum(D, tc_tiling=(D % 128 == 0))
  zD = jnp.zeros((_NP, D), jnp.float32)
  return k(y, src_p, dst_p, zD)


def _sc_degree(dst_p):
  k = _make_sc_degree()
  zD = jnp.zeros((_NP, 128), jnp.float32)
  ones = jnp.ones((_CH, 128), jnp.float32)
  return k(dst_p, zD, ones)


def _tc_proj2(x, Wl, Wr, bl):
  N, Din = x.shape
  Dl, Dr = Wl.shape[1], Wr.shape[1]

  def body(x_ref, wl_ref, wr_ref, b_ref, y_ref, z_ref):
    xb = x_ref[...]
    y_ref[...] = jnp.dot(xb, wl_ref[...], preferred_element_type=jnp.float32)
    z_ref[...] = (jnp.dot(xb, wr_ref[...], preferred_element_type=jnp.float32)
                  + b_ref[...])

  return pl.pallas_call(
      body,
      grid=(N // _BM,),
      in_specs=[pl.BlockSpec((_BM, Din), lambda i: (i, 0)),
                pl.BlockSpec((Din, Dl), lambda i: (0, 0)),
                pl.BlockSpec((Din, Dr), lambda i: (0, 0)),
                pl.BlockSpec((1, Dr), lambda i: (0, 0))],
      out_specs=[pl.BlockSpec((_BM, Dl), lambda i: (i, 0)),
                 pl.BlockSpec((_BM, Dr), lambda i: (i, 0))],
      out_shape=[jax.ShapeDtypeStruct((N, Dl), jnp.float32),
                 jax.ShapeDtypeStruct((N, Dr), jnp.float32)],
  )(x, Wl, Wr, bl.reshape(1, Dr))


def _tc_combine_proj(aggp, cntp, z1, Wl, Wr, b):
  N, H = z1.shape
  Do = Wl.shape[1]

  def body(a_ref, c_ref, z_ref, wl_ref, wr_ref, b_ref, y_ref, z2_ref):
    a = a_ref[0] + a_ref[1]
    cnt = c_ref[0, :, 0:1] + c_ref[1, :, 0:1]
    inv = 1.0 / jnp.maximum(cnt, 1.0)
    h = jnp.maximum(a * inv + z_ref[...], 0.0)
    y_ref[...] = jnp.dot(h, wl_ref[...], preferred_element_type=jnp.float32)
    z2_ref[...] = (jnp.dot(h, wr_ref[...], preferred_element_type=jnp.float32)
                   + b_ref[...])

  return pl.pallas_call(
      body,
      grid=(N // _BM,),
      in_specs=[pl.BlockSpec((2, _BM, H), lambda i: (0, i, 0)),
                pl.BlockSpec((2, _BM, 128), lambda i: (0, i, 0)),
                pl.BlockSpec((_BM, H), lambda i: (i, 0)),
                pl.BlockSpec((H, Do), lambda i: (0, 0)),
                pl.BlockSpec((H, Do), lambda i: (0, 0)),
                pl.BlockSpec((1, Do), lambda i: (0, 0))],
      out_specs=[pl.BlockSpec((_BM, Do), lambda i: (i, 0)),
                 pl.BlockSpec((_BM, Do), lambda i: (i, 0))],
      out_shape=[jax.ShapeDtypeStruct((N, Do), jnp.float32),
                 jax.ShapeDtypeStruct((N, Do), jnp.float32)],
  )(aggp, cntp, z1, Wl, Wr, b.reshape(1, Do))


def _tc_final(aggp, cntp, z2):
  N, Do = z2.shape

  def body(a_ref, c_ref, z_ref, o_ref):
    a = a_ref[0] + a_ref[1]
    cnt = c_ref[0, :, 0:1] + c_ref[1, :, 0:1]
    inv = 1.0 / jnp.maximum(cnt, 1.0)
    o_ref[...] = a * inv + z_ref[...]

  return pl.pallas_call(
      body,
      grid=(N // _BM,),
      in_specs=[pl.BlockSpec((2, _BM, Do), lambda i: (0, i, 0)),
                pl.BlockSpec((2, _BM, 128), lambda i: (0, i, 0)),
                pl.BlockSpec((_BM, Do), lambda i: (i, 0))],
      out_specs=pl.BlockSpec((_BM, Do), lambda i: (i, 0)),
      out_shape=jax.ShapeDtypeStruct((N, Do), jnp.float32),
  )(aggp, cntp, z2)


def kernel(x, edge_index, W1l, b1l, W1r, W2l, b2l, W2r):
  N, _ = x.shape
  C = W2l.shape[1]
  Dp = 64
  src = _pad_edges(edge_index[0].astype(jnp.int32), 0, _CH)
  dst = _pad_edges(edge_index[1].astype(jnp.int32), N, _NP - N)

  cntp = _sc_degree(dst)
  y1, z1 = _tc_proj2(x, W1l, W1r, b1l)
  agg1p = _sc_segsum(y1, src, dst)

  W2l_p = jnp.zeros((W2l.shape[0], Dp), jnp.float32).at[:, :C].set(W2l)
  W2r_p = jnp.zeros((W2r.shape[0], Dp), jnp.float32).at[:, :C].set(W2r)
  b2_p = jnp.zeros((Dp,), jnp.float32).at[:C].set(b2l)

  y2, z2 = _tc_combine_proj(agg1p, cntp, z1, W2l_p, W2r_p, b2_p)
  agg2p = _sc_segsum(y2, src, dst)
  out = _tc_final(agg2p, cntp, z2)
  return out[:, :C]


# speedup vs baseline: 2.8813x; 1.1836x over previous
import functools

import jax
import jax.numpy as jnp
from jax import lax
from jax.experimental import pallas as pl
from jax.experimental.pallas import tpu as pltpu
from jax.experimental.pallas import tpu_sc as plsc

_NC = 2
_NS = 16
_NW = _NC * _NS
_CH = 128
_NP = 10240
_BM = 2000
_U = 10
_STEPS = 80
_EPAD = _NW * _STEPS * _CH


@functools.lru_cache(maxsize=None)
def _make_sc_segsum(D, tc_tiling=True):
  rpt = _NP // _NS
  mesh = plsc.VectorSubcoreMesh(core_axis_name="c", subcore_axis_name="s")
  out_type = jax.ShapeDtypeStruct((_NC, _NP, D), jnp.float32)
  scratch = (pltpu.VMEM_SHARED((_NP, D), jnp.float32),
             pltpu.VMEM((_U, _CH), jnp.int32),
             pltpu.VMEM((_U, _CH), jnp.int32),
             pltpu.VMEM((_CH, D), jnp.float32),
             pltpu.VMEM((_CH, D), jnp.float32),
             pltpu.SemaphoreType.DMA,
             pltpu.SemaphoreType.DMA,
             pltpu.SemaphoreType.DMA,
             pltpu.SemaphoreType.DMA,
             pltpu.SemaphoreType.DMA)

  def body(y_h, src_h, dst_h, zD_h, agg_h,
           acc, si, di, rows0, rows1, semis, semid, semg, sems0, sems1):
    rows = (rows0, rows1)
    sems = (sems0, sems1)
    c = lax.axis_index("c")
    s = lax.axis_index("s")
    wid = c * _NS + s
    r0 = s * rpt
    pltpu.sync_copy(zD_h.at[pl.ds(r0, rpt)], acc.at[pl.ds(r0, rpt)])
    plsc.subcore_barrier()
    base = wid * _STEPS * _CH

    def outer(t, carry):
      bb = pl.multiple_of(base + t * _U * _CH, 8)
      idx_descs = []
      for k in range(_U):
        off = pl.multiple_of(bb + k * _CH, 8)
        idx_descs.append(
            pltpu.async_copy(src_h.at[pl.ds(off, _CH)], si.at[k], semis))
        idx_descs.append(
            pltpu.async_copy(dst_h.at[pl.ds(off, _CH)], di.at[k], semid))
      for d in idx_descs:
        d.wait()
      scat = [None, None]
      for k in range(_U):
        b2 = k % 2
        pltpu.async_copy(y_h.at[si.at[k]], rows[b2], semg).wait()
        if k >= 1:
          scat[1 - b2].wait()
        scat[b2] = pltpu.async_copy(rows[b2], acc.at[di.at[k]],
                                    sems[b2], add=True)
      scat[(_U - 1) % 2].wait()
      return carry

    lax.fori_loop(0, _STEPS // _U, outer, 0)
    plsc.subcore_barrier()
    pltpu.sync_copy(acc.at[pl.ds(r0, rpt)], agg_h.at[c, pl.ds(r0, rpt)])

  cp = None if tc_tiling else pltpu.CompilerParams(use_tc_tiling_on_sc=False)
  return pl.kernel(body, out_type=out_type, mesh=mesh, scratch_types=scratch,
                   compiler_params=cp)


_DC = 16


@functools.lru_cache(maxsize=None)
def _make_sc_degree():
  rpt = _NP // _NS
  mesh = plsc.VectorSubcoreMesh(core_axis_name="c", subcore_axis_name="s")
  out_type = jax.ShapeDtypeStruct((_NC, _NP, _DC), jnp.float32)
  scratch = (pltpu.VMEM_SHARED((_NP, _DC), jnp.float32),
             pltpu.VMEM((_CH, _DC), jnp.float32),
             pltpu.VMEM((_U, _CH), jnp.int32),
             pltpu.SemaphoreType.DMA,
             pltpu.SemaphoreType.DMA,
             pltpu.SemaphoreType.DMA)

  def body(dst_h, zD_h, ones_h, cnt_h, acc, ones, di, semid, sems0, sems1):
    sems = (sems0, sems1)
    c = lax.axis_index("c")
    s = lax.axis_index("s")
    wid = c * _NS + s
    r0 = s * rpt
    pltpu.sync_copy(zD_h.at[pl.ds(r0, rpt)], acc.at[pl.ds(r0, rpt)])
    pltpu.sync_copy(ones_h, ones)
    plsc.subcore_barrier()
    base = wid * _STEPS * _CH

    def outer(t, carry):
      bb = pl.multiple_of(base + t * _U * _CH, 8)
      idx_descs = []
      for k in range(_U):
        off = pl.multiple_of(bb + k * _CH, 8)
        idx_descs.append(
            pltpu.async_copy(dst_h.at[pl.ds(off, _CH)], di.at[k], semid))
      for d in idx_descs:
        d.wait()
      scat = [None, None]
      for k in range(_U):
        b2 = k % 2
        if k >= 2:
          scat[b2].wait()
        scat[b2] = pltpu.async_copy(ones, acc.at[di.at[k]],
                                    sems[b2], add=True)
      scat[0].wait()
      scat[1].wait()
      return carry

    lax.fori_loop(0, _STEPS // _U, outer, 0)
    plsc.subcore_barrier()
    pltpu.sync_copy(acc.at[pl.ds(r0, rpt)], cnt_h.at[c, pl.ds(r0, rpt)])

  return pl.kernel(body, out_type=out_type, mesh=mesh, scratch_types=scratch,
                   compiler_params=pltpu.CompilerParams(
                       use_tc_tiling_on_sc=False))


def _pad_edges(idx, base, mod):
  npad = _EPAD - idx.shape[0]
  fill = base + jnp.arange(npad, dtype=jnp.int32) % mod
  return jnp.concatenate([idx, fill])


def _sc_segsum(y, src_p, dst_p):
  _, D = y.shape
  k = _make_sc_segsum(D, tc_tiling=(D % 128 == 0))
  zD = jnp.zeros((_NP, D), jnp.float32)
  return k(y, src_p, dst_p, zD)


def _sc_degree(dst_p):
  k = _make_sc_degree()
  zD = jnp.zeros((_NP, _DC), jnp.float32)
  ones = jnp.ones((_CH, _DC), jnp.float32)
  return k(dst_p, zD, ones)


def _tc_proj2(x, Wl, Wr, bl):
  N, Din = x.shape
  Dl, Dr = Wl.shape[1], Wr.shape[1]

  def body(x_ref, wl_ref, wr_ref, b_ref, y_ref, z_ref):
    xb = x_ref[...]
    y_ref[...] = jnp.dot(xb, wl_ref[...], preferred_element_type=jnp.float32)
    z_ref[...] = (jnp.dot(xb, wr_ref[...], preferred_element_type=jnp.float32)
                  + b_ref[...])

  return pl.pallas_call(
      body,
      grid=(N // _BM,),
      in_specs=[pl.BlockSpec((_BM, Din), lambda i: (i, 0)),
                pl.BlockSpec((Din, Dl), lambda i: (0, 0)),
                pl.BlockSpec((Din, Dr), lambda i: (0, 0)),
                pl.BlockSpec((1, Dr), lambda i: (0, 0))],
      out_specs=[pl.BlockSpec((_BM, Dl), lambda i: (i, 0)),
                 pl.BlockSpec((_BM, Dr), lambda i: (i, 0))],
      out_shape=[jax.ShapeDtypeStruct((N, Dl), jnp.float32),
                 jax.ShapeDtypeStruct((N, Dr), jnp.float32)],
  )(x, Wl, Wr, bl.reshape(1, Dr))


def _tc_combine_proj(aggp, cntp, z1, Wl, Wr, b):
  N, H = z1.shape
  Do = Wl.shape[1]

  def body(a_ref, c_ref, z_ref, wl_ref, wr_ref, b_ref, y_ref, z2_ref):
    a = a_ref[0] + a_ref[1]
    cnt = c_ref[0, :, 0:1] + c_ref[1, :, 0:1]
    inv = 1.0 / jnp.maximum(cnt, 1.0)
    h = jnp.maximum(a * inv + z_ref[...], 0.0)
    y_ref[...] = jnp.dot(h, wl_ref[...], preferred_element_type=jnp.float32)
    z2_ref[...] = (jnp.dot(h, wr_ref[...], preferred_element_type=jnp.float32)
                   + b_ref[...])

  return pl.pallas_call(
      body,
      grid=(N // _BM,),
      in_specs=[pl.BlockSpec((2, _BM, H), lambda i: (0, i, 0)),
                pl.BlockSpec((2, _BM, _DC), lambda i: (0, i, 0)),
                pl.BlockSpec((_BM, H), lambda i: (i, 0)),
                pl.BlockSpec((H, Do), lambda i: (0, 0)),
                pl.BlockSpec((H, Do), lambda i: (0, 0)),
                pl.BlockSpec((1, Do), lambda i: (0, 0))],
      out_specs=[pl.BlockSpec((_BM, Do), lambda i: (i, 0)),
                 pl.BlockSpec((_BM, Do), lambda i: (i, 0))],
      out_shape=[jax.ShapeDtypeStruct((N, Do), jnp.float32),
                 jax.ShapeDtypeStruct((N, Do), jnp.float32)],
  )(aggp, cntp, z1, Wl, Wr, b.reshape(1, Do))


def _tc_final(aggp, cntp, z2):
  N, Do = z2.shape

  def body(a_ref, c_ref, z_ref, o_ref):
    a = a_ref[0] + a_ref[1]
    cnt = c_ref[0, :, 0:1] + c_ref[1, :, 0:1]
    inv = 1.0 / jnp.maximum(cnt, 1.0)
    o_ref[...] = a * inv + z_ref[...]

  return pl.pallas_call(
      body,
      grid=(N // _BM,),
      in_specs=[pl.BlockSpec((2, _BM, Do), lambda i: (0, i, 0)),
                pl.BlockSpec((2, _BM, _DC), lambda i: (0, i, 0)),
                pl.BlockSpec((_BM, Do), lambda i: (i, 0))],
      out_specs=pl.BlockSpec((_BM, Do), lambda i: (i, 0)),
      out_shape=jax.ShapeDtypeStruct((N, Do), jnp.float32),
  )(aggp, cntp, z2)


def kernel(x, edge_index, W1l, b1l, W1r, W2l, b2l, W2r):
  N, _ = x.shape
  C = W2l.shape[1]
  Dp = 64
  src = _pad_edges(edge_index[0].astype(jnp.int32), 0, _CH)
  dst = _pad_edges(edge_index[1].astype(jnp.int32), N, _NP - N)

  cntp = _sc_degree(dst)
  y1, z1 = _tc_proj2(x, W1l, W1r, b1l)
  agg1p = _sc_segsum(y1, src, dst)

  W2l_p = jnp.zeros((W2l.shape[0], Dp), jnp.float32).at[:, :C].set(W2l)
  W2r_p = jnp.zeros((W2r.shape[0], Dp), jnp.float32).at[:, :C].set(W2r)
  b2_p = jnp.zeros((Dp,), jnp.float32).at[:C].set(b2l)

  y2, z2 = _tc_combine_proj(agg1p, cntp, z1, W2l_p, W2r_p, b2_p)
  agg2p = _sc_segsum(y2, src, dst)
  out = _tc_final(agg2p, cntp, z2)
  return out[:, :C]


# speedup vs baseline: 2.9425x; 1.0212x over previous
import functools

import jax
import jax.numpy as jnp
from jax import lax
from jax.experimental import pallas as pl
from jax.experimental.pallas import tpu as pltpu
from jax.experimental.pallas import tpu_sc as plsc

_NC = 2
_NS = 16
_NW = _NC * _NS
_CH = 80
_NP = 10240
_BM = 2000
_U = 8
_STEPS = 128
_EPAD = _NW * _STEPS * _CH


@functools.lru_cache(maxsize=None)
def _make_sc_segsum(D, tc_tiling=True):
  rpt = _NP // _NS
  mesh = plsc.VectorSubcoreMesh(core_axis_name="c", subcore_axis_name="s")
  out_type = jax.ShapeDtypeStruct((_NC, _NP, D), jnp.float32)
  scratch = (pltpu.VMEM_SHARED((_NP, D), jnp.float32),
             pltpu.VMEM((_U, _CH), jnp.int32),
             pltpu.VMEM((_U, _CH), jnp.int32),
             pltpu.VMEM((_CH, D), jnp.float32),
             pltpu.VMEM((_CH, D), jnp.float32),
             pltpu.VMEM((_CH, D), jnp.float32),
             pltpu.SemaphoreType.DMA,
             pltpu.SemaphoreType.DMA,
             pltpu.SemaphoreType.DMA,
             pltpu.SemaphoreType.DMA,
             pltpu.SemaphoreType.DMA,
             pltpu.SemaphoreType.DMA,
             pltpu.SemaphoreType.DMA,
             pltpu.SemaphoreType.DMA)

  def body(y_h, src_h, dst_h, zD_h, agg_h,
           acc, si, di, rows0, rows1, rows2, semis, semid,
           semg0, semg1, semg2, sems0, sems1, sems2):
    rows = (rows0, rows1, rows2)
    semg = (semg0, semg1, semg2)
    sems = (sems0, sems1, sems2)
    c = lax.axis_index("c")
    s = lax.axis_index("s")
    wid = c * _NS + s
    r0 = s * rpt
    pltpu.sync_copy(zD_h.at[pl.ds(r0, rpt)], acc.at[pl.ds(r0, rpt)])
    plsc.subcore_barrier()
    base = wid * _STEPS * _CH

    def outer(t, carry):
      bb = pl.multiple_of(base + t * _U * _CH, 8)
      idx_descs = []
      for k in range(_U):
        off = pl.multiple_of(bb + k * _CH, 8)
        idx_descs.append(
            pltpu.async_copy(src_h.at[pl.ds(off, _CH)], si.at[k], semis))
        idx_descs.append(
            pltpu.async_copy(dst_h.at[pl.ds(off, _CH)], di.at[k], semid))
      for d in idx_descs:
        d.wait()
      def gath(k):
        b3 = k % 3
        return pltpu.async_copy(y_h.at[si.at[k]], rows[b3], semg[b3])
      gd = [None, None, None]
      scat = [None, None, None]
      gd[0] = gath(0)
      for k in range(_U):
        b3 = k % 3
        if k + 1 < _U:
          nb = (k + 1) % 3
          if k + 1 >= 3:
            scat[nb].wait()
          gd[nb] = gath(k + 1)
        gd[b3].wait()
        scat[b3] = pltpu.async_copy(rows[b3], acc.at[di.at[k]],
                                    sems[b3], add=True)
      for j in range(3):
        scat[(_U - 3 + j) % 3].wait()
      return carry

    lax.fori_loop(0, _STEPS // _U, outer, 0)
    plsc.subcore_barrier()
    pltpu.sync_copy(acc.at[pl.ds(r0, rpt)], agg_h.at[c, pl.ds(r0, rpt)])

  cp = None if tc_tiling else pltpu.CompilerParams(use_tc_tiling_on_sc=False)
  return pl.kernel(body, out_type=out_type, mesh=mesh, scratch_types=scratch,
                   compiler_params=cp)


_DC = 16


@functools.lru_cache(maxsize=None)
def _make_sc_degree():
  rpt = _NP // _NS
  mesh = plsc.VectorSubcoreMesh(core_axis_name="c", subcore_axis_name="s")
  out_type = jax.ShapeDtypeStruct((_NC, _NP, _DC), jnp.float32)
  scratch = (pltpu.VMEM_SHARED((_NP, _DC), jnp.float32),
             pltpu.VMEM((_CH, _DC), jnp.float32),
             pltpu.VMEM((_U, _CH), jnp.int32),
             pltpu.SemaphoreType.DMA,
             pltpu.SemaphoreType.DMA,
             pltpu.SemaphoreType.DMA)

  def body(dst_h, zD_h, ones_h, cnt_h, acc, ones, di, semid, sems0, sems1):
    sems = (sems0, sems1)
    c = lax.axis_index("c")
    s = lax.axis_index("s")
    wid = c * _NS + s
    r0 = s * rpt
    pltpu.sync_copy(zD_h.at[pl.ds(r0, rpt)], acc.at[pl.ds(r0, rpt)])
    pltpu.sync_copy(ones_h, ones)
    plsc.subcore_barrier()
    base = wid * _STEPS * _CH

    def outer(t, carry):
      bb = pl.multiple_of(base + t * _U * _CH, 8)
      idx_descs = []
      for k in range(_U):
        off = pl.multiple_of(bb + k * _CH, 8)
        idx_descs.append(
            pltpu.async_copy(dst_h.at[pl.ds(off, _CH)], di.at[k], semid))
      for d in idx_descs:
        d.wait()
      scat = [None, None]
      for k in range(_U):
        b2 = k % 2
        if k >= 2:
          scat[b2].wait()
        scat[b2] = pltpu.async_copy(ones, acc.at[di.at[k]],
                                    sems[b2], add=True)
      scat[0].wait()
      scat[1].wait()
      return carry

    lax.fori_loop(0, _STEPS // _U, outer, 0)
    plsc.subcore_barrier()
    pltpu.sync_copy(acc.at[pl.ds(r0, rpt)], cnt_h.at[c, pl.ds(r0, rpt)])

  return pl.kernel(body, out_type=out_type, mesh=mesh, scratch_types=scratch,
                   compiler_params=pltpu.CompilerParams(
                       use_tc_tiling_on_sc=False))


def _pad_edges(idx, base, mod):
  npad = _EPAD - idx.shape[0]
  fill = base + jnp.arange(npad, dtype=jnp.int32) % mod
  return jnp.concatenate([idx, fill])


def _sc_segsum(y, src_p, dst_p):
  _, D = y.shape
  k = _make_sc_segsum(D, tc_tiling=(D % 128 == 0))
  zD = jnp.zeros((_NP, D), jnp.float32)
  return k(y, src_p, dst_p, zD)


def _sc_degree(dst_p):
  k = _make_sc_degree()
  zD = jnp.zeros((_NP, _DC), jnp.float32)
  ones = jnp.ones((_CH, _DC), jnp.float32)
  return k(dst_p, zD, ones)


def _tc_proj2(x, Wl, Wr, bl):
  N, Din = x.shape
  Dl, Dr = Wl.shape[1], Wr.shape[1]

  def body(x_ref, wl_ref, wr_ref, b_ref, y_ref, z_ref):
    xb = x_ref[...]
    y_ref[...] = jnp.dot(xb, wl_ref[...], preferred_element_type=jnp.float32)
    z_ref[...] = (jnp.dot(xb, wr_ref[...], preferred_element_type=jnp.float32)
                  + b_ref[...])

  return pl.pallas_call(
      body,
      grid=(N // _BM,),
      in_specs=[pl.BlockSpec((_BM, Din), lambda i: (i, 0)),
                pl.BlockSpec((Din, Dl), lambda i: (0, 0)),
                pl.BlockSpec((Din, Dr), lambda i: (0, 0)),
                pl.BlockSpec((1, Dr), lambda i: (0, 0))],
      out_specs=[pl.BlockSpec((_BM, Dl), lambda i: (i, 0)),
                 pl.BlockSpec((_BM, Dr), lambda i: (i, 0))],
      out_shape=[jax.ShapeDtypeStruct((N, Dl), jnp.float32),
                 jax.ShapeDtypeStruct((N, Dr), jnp.float32)],
  )(x, Wl, Wr, bl.reshape(1, Dr))


def _tc_combine_proj(aggp, cntp, z1, Wl, Wr, b):
  N, H = z1.shape
  Do = Wl.shape[1]

  def body(a_ref, c_ref, z_ref, wl_ref, wr_ref, b_ref, y_ref, z2_ref):
    a = a_ref[0] + a_ref[1]
    cnt = c_ref[0, :, 0:1] + c_ref[1, :, 0:1]
    inv = 1.0 / jnp.maximum(cnt, 1.0)
    h = jnp.maximum(a * inv + z_ref[...], 0.0)
    y_ref[...] = jnp.dot(h, wl_ref[...], preferred_element_type=jnp.float32)
    z2_ref[...] = (jnp.dot(h, wr_ref[...], preferred_element_type=jnp.float32)
                   + b_ref[...])

  return pl.pallas_call(
      body,
      grid=(N // _BM,),
      in_specs=[pl.BlockSpec((2, _BM, H), lambda i: (0, i, 0)),
                pl.BlockSpec((2, _BM, _DC), lambda i: (0, i, 0)),
                pl.BlockSpec((_BM, H), lambda i: (i, 0)),
                pl.BlockSpec((H, Do), lambda i: (0, 0)),
                pl.BlockSpec((H, Do), lambda i: (0, 0)),
                pl.BlockSpec((1, Do), lambda i: (0, 0))],
      out_specs=[pl.BlockSpec((_BM, Do), lambda i: (i, 0)),
                 pl.BlockSpec((_BM, Do), lambda i: (i, 0))],
      out_shape=[jax.ShapeDtypeStruct((N, Do), jnp.float32),
                 jax.ShapeDtypeStruct((N, Do), jnp.float32)],
  )(aggp, cntp, z1, Wl, Wr, b.reshape(1, Do))


def _tc_final(aggp, cntp, z2):
  N, Do = z2.shape

  def body(a_ref, c_ref, z_ref, o_ref):
    a = a_ref[0] + a_ref[1]
    cnt = c_ref[0, :, 0:1] + c_ref[1, :, 0:1]
    inv = 1.0 / jnp.maximum(cnt, 1.0)
    o_ref[...] = a * inv + z_ref[...]

  return pl.pallas_call(
      body,
      grid=(N // _BM,),
      in_specs=[pl.BlockSpec((2, _BM, Do), lambda i: (0, i, 0)),
                pl.BlockSpec((2, _BM, _DC), lambda i: (0, i, 0)),
                pl.BlockSpec((_BM, Do), lambda i: (i, 0))],
      out_specs=pl.BlockSpec((_BM, Do), lambda i: (i, 0)),
      out_shape=jax.ShapeDtypeStruct((N, Do), jnp.float32),
  )(aggp, cntp, z2)


def kernel(x, edge_index, W1l, b1l, W1r, W2l, b2l, W2r):
  N, _ = x.shape
  C = W2l.shape[1]
  Dp = 64
  src = _pad_edges(edge_index[0].astype(jnp.int32), 0, _CH)
  dst = _pad_edges(edge_index[1].astype(jnp.int32), N, _NP - N)

  cntp = _sc_degree(dst)
  y1, z1 = _tc_proj2(x, W1l, W1r, b1l)
  agg1p = _sc_segsum(y1, src, dst)

  W2l_p = jnp.zeros((W2l.shape[0], Dp), jnp.float32).at[:, :C].set(W2l)
  W2r_p = jnp.zeros((W2r.shape[0], Dp), jnp.float32).at[:, :C].set(W2r)
  b2_p = jnp.zeros((Dp,), jnp.float32).at[:C].set(b2l)

  y2, z2 = _tc_combine_proj(agg1p, cntp, z1, W2l_p, W2r_p, b2_p)
  agg2p = _sc_segsum(y2, src, dst)
  out = _tc_final(agg2p, cntp, z2)
  return out[:, :C]


# speedup vs baseline: 3.1070x; 1.0559x over previous
import functools

import jax
import jax.numpy as jnp
from jax import lax
from jax.experimental import pallas as pl
from jax.experimental.pallas import tpu as pltpu
from jax.experimental.pallas import tpu_sc as plsc

_NC = 2
_NS = 16
_NW = _NC * _NS
_CH = 80
_NP = 10240
_BM = 2000
_U = 8
_STEPS = 128
_EPAD = _NW * _STEPS * _CH


@functools.lru_cache(maxsize=None)
def _make_sc_segsum(D, with_cnt=False):
  rpt = _NP // _NS
  mesh = plsc.VectorSubcoreMesh(core_axis_name="c", subcore_axis_name="s")
  out_type = jax.ShapeDtypeStruct((_NC, _NP, D), jnp.float32)
  if with_cnt:
    out_type = (out_type, jax.ShapeDtypeStruct((_NC, _NP, _DC), jnp.float32))
  scratch = ((pltpu.VMEM_SHARED((_NP, D), jnp.float32),)
             + ((pltpu.VMEM_SHARED((_NP, _DC), jnp.float32),
                 pltpu.VMEM((_CH, _DC), jnp.float32)) if with_cnt else ())
             + (pltpu.VMEM((_U, _CH), jnp.int32),
                pltpu.VMEM((_U, _CH), jnp.int32),
                pltpu.VMEM((_CH, D), jnp.float32),
                pltpu.VMEM((_CH, D), jnp.float32),
                pltpu.VMEM((_CH, D), jnp.float32))
             + tuple(pltpu.SemaphoreType.DMA
                     for _ in range(8 + (3 if with_cnt else 0))))

  def body(*args):
    if with_cnt:
      (y_h, src_h, dst_h, zD_h, zC_h, ones_h, agg_h, cnt_h,
       acc, acc16, ones, si, di, rows0, rows1, rows2, semis, semid,
       semg0, semg1, semg2, sems0, sems1, sems2, semo0, semo1, semo2) = args
      semo = (semo0, semo1, semo2)
    else:
      (y_h, src_h, dst_h, zD_h, agg_h,
       acc, si, di, rows0, rows1, rows2, semis, semid,
       semg0, semg1, semg2, sems0, sems1, sems2) = args
    rows = (rows0, rows1, rows2)
    semg = (semg0, semg1, semg2)
    sems = (sems0, sems1, sems2)
    c = lax.axis_index("c")
    s = lax.axis_index("s")
    wid = c * _NS + s
    r0 = s * rpt
    pltpu.sync_copy(zD_h.at[pl.ds(r0, rpt)], acc.at[pl.ds(r0, rpt)])
    if with_cnt:
      pltpu.sync_copy(zC_h.at[pl.ds(r0, rpt)], acc16.at[pl.ds(r0, rpt)])
      pltpu.sync_copy(ones_h, ones)
    plsc.subcore_barrier()
    base = wid * _STEPS * _CH

    def outer(t, carry):
      bb = pl.multiple_of(base + t * _U * _CH, 8)
      idx_descs = []
      for k in range(_U):
        off = pl.multiple_of(bb + k * _CH, 8)
        idx_descs.append(
            pltpu.async_copy(src_h.at[pl.ds(off, _CH)], si.at[k], semis))
        idx_descs.append(
            pltpu.async_copy(dst_h.at[pl.ds(off, _CH)], di.at[k], semid))
      for d in idx_descs:
        d.wait()
      def gath(k):
        b3 = k % 3
        return pltpu.async_copy(y_h.at[si.at[k]], rows[b3], semg[b3])
      gd = [None, None, None]
      scat = [None, None, None]
      oscat = [None, None, None]
      gd[0] = gath(0)
      for k in range(_U):
        b3 = k % 3
        if k + 1 < _U:
          nb = (k + 1) % 3
          if k + 1 >= 3:
            scat[nb].wait()
            if with_cnt:
              oscat[nb].wait()
          gd[nb] = gath(k + 1)
        gd[b3].wait()
        scat[b3] = pltpu.async_copy(rows[b3], acc.at[di.at[k]],
                                    sems[b3], add=True)
        if with_cnt:
          oscat[b3] = pltpu.async_copy(ones, acc16.at[di.at[k]],
                                       semo[b3], add=True)
      for j in range(3):
        scat[(_U - 3 + j) % 3].wait()
        if with_cnt:
          oscat[(_U - 3 + j) % 3].wait()
      return carry

    lax.fori_loop(0, _STEPS // _U, outer, 0)
    plsc.subcore_barrier()
    pltpu.sync_copy(acc.at[pl.ds(r0, rpt)], agg_h.at[c, pl.ds(r0, rpt)])
    if with_cnt:
      pltpu.sync_copy(acc16.at[pl.ds(r0, rpt)], cnt_h.at[c, pl.ds(r0, rpt)])

  return pl.kernel(body, out_type=out_type, mesh=mesh, scratch_types=scratch,
                   compiler_params=pltpu.CompilerParams(
                       use_tc_tiling_on_sc=False))


_DC = 16


def _pad_edges(idx, base, mod):
  npad = _EPAD - idx.shape[0]
  fill = base + jnp.arange(npad, dtype=jnp.int32) % mod
  return jnp.concatenate([idx, fill])


def _sc_segsum(y, src_p, dst_p):
  _, D = y.shape
  k = _make_sc_segsum(D)
  zD = jnp.zeros((_NP, D), jnp.float32)
  return k(y, src_p, dst_p, zD)


def _sc_segsum_cnt(y, src_p, dst_p):
  _, D = y.shape
  k = _make_sc_segsum(D, with_cnt=True)
  zD = jnp.zeros((_NP, D), jnp.float32)
  zC = jnp.zeros((_NP, _DC), jnp.float32)
  ones = jnp.ones((_CH, _DC), jnp.float32)
  return k(y, src_p, dst_p, zD, zC, ones)


def _tc_proj2(x, Wl, Wr, bl):
  N, Din = x.shape
  Dl, Dr = Wl.shape[1], Wr.shape[1]

  def body(x_ref, wl_ref, wr_ref, b_ref, y_ref, z_ref):
    xb = x_ref[...]
    y_ref[...] = jnp.dot(xb, wl_ref[...], preferred_element_type=jnp.float32)
    z_ref[...] = (jnp.dot(xb, wr_ref[...], preferred_element_type=jnp.float32)
                  + b_ref[...])

  return pl.pallas_call(
      body,
      grid=(N // _BM,),
      in_specs=[pl.BlockSpec((_BM, Din), lambda i: (i, 0)),
                pl.BlockSpec((Din, Dl), lambda i: (0, 0)),
                pl.BlockSpec((Din, Dr), lambda i: (0, 0)),
                pl.BlockSpec((1, Dr), lambda i: (0, 0))],
      out_specs=[pl.BlockSpec((_BM, Dl), lambda i: (i, 0)),
                 pl.BlockSpec((_BM, Dr), lambda i: (i, 0))],
      out_shape=[jax.ShapeDtypeStruct((N, Dl), jnp.float32),
                 jax.ShapeDtypeStruct((N, Dr), jnp.float32)],
  )(x, Wl, Wr, bl.reshape(1, Dr))


def _tc_combine_proj(aggp, cntp, z1, Wl, Wr, b):
  N, H = z1.shape
  Do = Wl.shape[1]

  def body(a_ref, c_ref, z_ref, wl_ref, wr_ref, b_ref, y_ref, z2_ref):
    a = a_ref[0] + a_ref[1]
    cnt = c_ref[0, :, 0:1] + c_ref[1, :, 0:1]
    inv = 1.0 / jnp.maximum(cnt, 1.0)
    h = jnp.maximum(a * inv + z_ref[...], 0.0)
    y_ref[...] = jnp.dot(h, wl_ref[...], preferred_element_type=jnp.float32)
    z2_ref[...] = (jnp.dot(h, wr_ref[...], preferred_element_type=jnp.float32)
                   + b_ref[...])

  return pl.pallas_call(
      body,
      grid=(N // _BM,),
      in_specs=[pl.BlockSpec((2, _BM, H), lambda i: (0, i, 0)),
                pl.BlockSpec((2, _BM, _DC), lambda i: (0, i, 0)),
                pl.BlockSpec((_BM, H), lambda i: (i, 0)),
                pl.BlockSpec((H, Do), lambda i: (0, 0)),
                pl.BlockSpec((H, Do), lambda i: (0, 0)),
                pl.BlockSpec((1, Do), lambda i: (0, 0))],
      out_specs=[pl.BlockSpec((_BM, Do), lambda i: (i, 0)),
                 pl.BlockSpec((_BM, Do), lambda i: (i, 0))],
      out_shape=[jax.ShapeDtypeStruct((N, Do), jnp.float32),
                 jax.ShapeDtypeStruct((N, Do), jnp.float32)],
  )(aggp, cntp, z1, Wl, Wr, b.reshape(1, Do))


def _tc_final(aggp, cntp, z2):
  N, Do = z2.shape

  def body(a_ref, c_ref, z_ref, o_ref):
    a = a_ref[0] + a_ref[1]
    cnt = c_ref[0, :, 0:1] + c_ref[1, :, 0:1]
    inv = 1.0 / jnp.maximum(cnt, 1.0)
    o_ref[...] = a * inv + z_ref[...]

  return pl.pallas_call(
      body,
      grid=(N // _BM,),
      in_specs=[pl.BlockSpec((2, _BM, Do), lambda i: (0, i, 0)),
                pl.BlockSpec((2, _BM, _DC), lambda i: (0, i, 0)),
                pl.BlockSpec((_BM, Do), lambda i: (i, 0))],
      out_specs=pl.BlockSpec((_BM, Do), lambda i: (i, 0)),
      out_shape=jax.ShapeDtypeStruct((N, Do), jnp.float32),
  )(aggp, cntp, z2)


def kernel(x, edge_index, W1l, b1l, W1r, W2l, b2l, W2r):
  N, _ = x.shape
  C = W2l.shape[1]
  Dp = 64
  src = _pad_edges(edge_index[0].astype(jnp.int32), 0, _CH)
  dst = _pad_edges(edge_index[1].astype(jnp.int32), N, _NP - N)

  y1, z1 = _tc_proj2(x, W1l, W1r, b1l)
  agg1p, cntp = _sc_segsum_cnt(y1, src, dst)

  W2l_p = jnp.zeros((W2l.shape[0], Dp), jnp.float32).at[:, :C].set(W2l)
  W2r_p = jnp.zeros((W2r.shape[0], Dp), jnp.float32).at[:, :C].set(W2r)
  b2_p = jnp.zeros((Dp,), jnp.float32).at[:C].set(b2l)

  y2, z2 = _tc_combine_proj(agg1p, cntp, z1, W2l_p, W2r_p, b2_p)
  agg2p = _sc_segsum(y2, src, dst)
  out = _tc_final(agg2p, cntp, z2)
  return out[:, :C]
